# Initial kernel scaffold; baseline (speedup 1.0000x reference)
import jax
import jax.numpy as jnp
from jax.experimental import pallas as pl


def kernel(x, edge_index, batch, W1l, W1r, b1, g1, be1, p1_wl, p1_wr, p1_b, W2l, W2r, b2, g2, be2, p2_wl, p2_wr, p2_b, Wlin, blin):
    raise NotImplementedError("write your pallas kernel here")


# speedup vs baseline: 4.2388x; 4.2388x over previous
import functools

import jax
import jax.numpy as jnp
from jax import lax
from jax.experimental import pallas as pl

N = 10000
NP = 10240
E = 320000
G = 64
F0 = 128
H1 = 256
H2 = 512
BI = 256
BK = 2048
NI = NP // BI
NJ = NP // BK
RATIO = 0.5
EPS = 1e-5
F32 = jnp.float32


def _k_amm(a_ref, x_ref, o_ref, *, nj):
    j = pl.program_id(1)
    part = jnp.dot(a_ref[...], x_ref[...], preferred_element_type=F32)

    @pl.when(j == 0)
    def _():
        o_ref[...] = part

    @pl.when(j > 0)
    def _():
        o_ref[...] += part


def _amm(A, Xa, width):
    return pl.pallas_call(
        functools.partial(_k_amm, nj=NJ),
        grid=(NI, NJ),
        in_specs=[
            pl.BlockSpec((BI, BK), lambda i, j: (i, j)),
            pl.BlockSpec((BK, width), lambda i, j: (j, 0)),
        ],
        out_specs=pl.BlockSpec((BI, width), lambda i, j: (i, 0)),
        out_shape=jax.ShapeDtypeStruct((NP, width), F32),
    )(A, Xa)


def _k_lin1(c_ref, x_ref, wl_ref, wr_ref, b_ref, z_ref, s_ref, q_ref, *, ni):
    i = pl.program_id(0)
    c = c_ref[...]
    aggr = c[:, :F0]
    deg = c[:, F0:F0 + 1]
    aggrn = aggr / jnp.maximum(deg, 1.0)
    z = (jnp.dot(aggrn, wl_ref[...], preferred_element_type=F32)
         + jnp.dot(x_ref[...], wr_ref[...], preferred_element_type=F32)
         + b_ref[...])
    row = i * BI + lax.broadcasted_iota(jnp.int32, (BI, 1), 0)
    z = jnp.where(row < N, z, 0.0)
    z_ref[...] = z

    @pl.when(i == 0)
    def _():
        s_ref[...] = jnp.zeros_like(s_ref)
        q_ref[...] = jnp.zeros_like(q_ref)

    s_ref[...] += jnp.sum(z, axis=0, keepdims=True)
    q_ref[...] += jnp.sum(z * z, axis=0, keepdims=True)


def _lin1(C1, xp, W1l, W1r, b1):
    return pl.pallas_call(
        functools.partial(_k_lin1, ni=NI),
        grid=(NI,),
        in_specs=[
            pl.BlockSpec((BI, H1), lambda i: (i, 0)),
            pl.BlockSpec((BI, F0), lambda i: (i, 0)),
            pl.BlockSpec((F0, H1), lambda i: (0, 0)),
            pl.BlockSpec((F0, H1), lambda i: (0, 0)),
            pl.BlockSpec((1, H1), lambda i: (0, 0)),
        ],
        out_specs=[
            pl.BlockSpec((BI, H1), lambda i: (i, 0)),
            pl.BlockSpec((1, H1), lambda i: (0, 0)),
            pl.BlockSpec((1, H1), lambda i: (0, 0)),
        ],
        out_shape=[
            jax.ShapeDtypeStruct((NP, H1), F32),
            jax.ShapeDtypeStruct((1, H1), F32),
            jax.ShapeDtypeStruct((1, H1), F32),
        ],
    )(C1, xp, W1l, W1r, b1)


def _k_bn1(z_ref, s_ref, q_ref, g_ref, be_ref, wl_ref, wr_ref,
           h_ref, v_ref, u_ref):
    mu = s_ref[...] / N
    var = q_ref[...] / N - mu * mu
    rstd = lax.rsqrt(var + EPS)
    h = jnp.maximum((z_ref[...] - mu) * rstd * g_ref[...] + be_ref[...], 0.0)
    h_ref[...] = h
    v_ref[...] = jnp.dot(h, wl_ref[...], preferred_element_type=F32)
    u_ref[...] = jnp.dot(h, wr_ref[...], preferred_element_type=F32)


def _bn1(Z1, S1, Q1, g1, be1, p1_wl, p1_wr):
    return pl.pallas_call(
        _k_bn1,
        grid=(NI,),
        in_specs=[
            pl.BlockSpec((BI, H1), lambda i: (i, 0)),
            pl.BlockSpec((1, H1), lambda i: (0, 0)),
            pl.BlockSpec((1, H1), lambda i: (0, 0)),
            pl.BlockSpec((1, H1), lambda i: (0, 0)),
            pl.BlockSpec((1, H1), lambda i: (0, 0)),
            pl.BlockSpec((H1, 1), lambda i: (0, 0)),
            pl.BlockSpec((H1, 1), lambda i: (0, 0)),
        ],
        out_specs=[
            pl.BlockSpec((BI, H1), lambda i: (i, 0)),
            pl.BlockSpec((BI, 1), lambda i: (i, 0)),
            pl.BlockSpec((BI, 1), lambda i: (i, 0)),
        ],
        out_shape=[
            jax.ShapeDtypeStruct((NP, H1), F32),
            jax.ShapeDtypeStruct((NP, 1), F32),
            jax.ShapeDtypeStruct((NP, 1), F32),
        ],
    )(Z1, S1, Q1, g1, be1, p1_wl, p1_wr)


def _k_matvec(a_ref, v_ref, u_ref, m_ref, b_ref, o_ref, *, nj):
    j = pl.program_id(1)
    part = jnp.dot(a_ref[...], v_ref[...], preferred_element_type=F32)

    @pl.when(j == 0)
    def _():
        o_ref[...] = part

    @pl.when(j > 0)
    def _():
        o_ref[...] += part

    @pl.when(j == nj - 1)
    def _():
        o_ref[...] = m_ref[...] * o_ref[...] + u_ref[...] + b_ref[...]


def _score(A, v, u, m, b):
    return pl.pallas_call(
        functools.partial(_k_matvec, nj=NJ),
        grid=(NI, NJ),
        in_specs=[
            pl.BlockSpec((BI, BK), lambda i, j: (i, j)),
            pl.BlockSpec((BK, 1), lambda i, j: (j, 0)),
            pl.BlockSpec((BI, 1), lambda i, j: (i, 0)),
            pl.BlockSpec((BI, 1), lambda i, j: (i, 0)),
            pl.BlockSpec((1, 1), lambda i, j: (0, 0)),
        ],
        out_specs=pl.BlockSpec((BI, 1), lambda i, j: (i, 0)),
        out_shape=jax.ShapeDtypeStruct((NP, 1), F32),
    )(A, v, u, m, b)


def _k_topk(sc_ref, tc_ref, bc_ref, mc_ref, sr_ref, tr_ref, br_ref, mr_ref,
            rank_ref, cnt_ref, keep_ref, *, nj):
    i = pl.program_id(0)
    j = pl.program_id(1)

    @pl.when(j == 0)
    def _():
        rank_ref[...] = jnp.zeros_like(rank_ref)
        cnt_ref[...] = jnp.zeros_like(cnt_ref)

    bi = bc_ref[...]
    bj = br_ref[0]
    si = sc_ref[...]
    sj = sr_ref[0]
    ti = tc_ref[...]
    tj = tr_ref[0]
    mj = mr_ref[0] > 0.0
    ii = i * BI + lax.broadcasted_iota(jnp.int32, (BI, 1), 0)
    jj = j * BK + lax.broadcasted_iota(jnp.int32, (1, BK), 1)
    sg = (bi == bj) & mj
    gt = sj > si
    eq = sj == si
    tgt = tj > ti
    teq = tj == ti
    prec = gt | (eq & (tgt | (teq & (jj < ii))))
    rank_ref[...] += jnp.sum(jnp.where(sg & prec, 1.0, 0.0), axis=1, keepdims=True)
    cnt_ref[...] += jnp.sum(jnp.where(sg, 1.0, 0.0), axis=1, keepdims=True)

    @pl.when(j == nj - 1)
    def _():
        k = jnp.ceil(RATIO * cnt_ref[...])
        keep_ref[...] = jnp.where((mc_ref[...] > 0.0) & (rank_ref[...] < k),
                                  1.0, 0.0)


def _topk(s, t, batch_c, m):
    s_r = s.reshape(NJ, 1, BK)
    t_r = t.reshape(NJ, 1, BK)
    b_r = batch_c.reshape(NJ, 1, BK)
    m_r = m.reshape(NJ, 1, BK)
    col = lambda i, j: (i, 0)
    row = lambda i, j: (j, 0, 0)
    outs = pl.pallas_call(
        functools.partial(_k_topk, nj=NJ),
        grid=(NI, NJ),
        in_specs=[
            pl.BlockSpec((BI, 1), col),
            pl.BlockSpec((BI, 1), col),
            pl.BlockSpec((BI, 1), col),
            pl.BlockSpec((BI, 1), col),
            pl.BlockSpec((1, 1, BK), row),
            pl.BlockSpec((1, 1, BK), row),
            pl.BlockSpec((1, 1, BK), row),
            pl.BlockSpec((1, 1, BK), row),
        ],
        out_specs=[
            pl.BlockSpec((BI, 1), col),
            pl.BlockSpec((BI, 1), col),
            pl.BlockSpec((BI, 1), col),
        ],
        out_shape=[
            jax.ShapeDtypeStruct((NP, 1), F32),
            jax.ShapeDtypeStruct((NP, 1), F32),
            jax.ShapeDtypeStruct((NP, 1), F32),
        ],
    )(s, t, batch_c, m, s_r, t_r, b_r, m_r)
    return outs[2]


def _k_aggr2(a_ref, h_ref, s1_ref, k1j_ref, k1i_ref, ag_ref, dm_ref, *, nj):
    j = pl.program_id(1)
    w = k1j_ref[...] * jnp.tanh(s1_ref[...])
    hm = h_ref[...] * w
    part = jnp.dot(a_ref[...], hm, preferred_element_type=F32)
    dpart = jnp.dot(a_ref[...], k1j_ref[...], preferred_element_type=F32)

    @pl.when(j == 0)
    def _():
        ag_ref[...] = part
        dm_ref[...] = dpart

    @pl.when(j > 0)
    def _():
        ag_ref[...] += part
        dm_ref[...] += dpart

    @pl.when(j == nj - 1)
    def _():
        mi = k1i_ref[...]
        degm = mi * dm_ref[...]
        ag_ref[...] = mi * ag_ref[...] / jnp.maximum(degm, 1.0)


def _aggr2(A, h, s1, keep1):
    return pl.pallas_call(
        functools.partial(_k_aggr2, nj=NJ),
        grid=(NI, NJ),
        in_specs=[
            pl.BlockSpec((BI, BK), lambda i, j: (i, j)),
            pl.BlockSpec((BK, H1), lambda i, j: (j, 0)),
            pl.BlockSpec((BK, 1), lambda i, j: (j, 0)),
            pl.BlockSpec((BK, 1), lambda i, j: (j, 0)),
            pl.BlockSpec((BI, 1), lambda i, j: (i, 0)),
        ],
        out_specs=[
            pl.BlockSpec((BI, H1), lambda i, j: (i, 0)),
            pl.BlockSpec((BI, 1), lambda i, j: (i, 0)),
        ],
        out_shape=[
            jax.ShapeDtypeStruct((NP, H1), F32),
            jax.ShapeDtypeStruct((NP, 1), F32),
        ],
    )(A, h, s1, keep1, keep1)


def _k_lin2(ag_ref, h_ref, s1_ref, k1_ref, wl_ref, wr_ref, b_ref,
            z_ref, s_ref, q_ref, c_ref):
    i = pl.program_id(0)
    k1 = k1_ref[...]
    hm = h_ref[...] * (k1 * jnp.tanh(s1_ref[...]))
    z = (jnp.dot(ag_ref[...], wl_ref[...], preferred_element_type=F32)
         + jnp.dot(hm, wr_ref[...], preferred_element_type=F32)
         + b_ref[...])
    z_ref[...] = z

    @pl.when(i == 0)
    def _():
        s_ref[...] = jnp.zeros_like(s_ref)
        q_ref[...] = jnp.zeros_like(q_ref)
        c_ref[...] = jnp.zeros_like(c_ref)

    s_ref[...] += jnp.sum(z * k1, axis=0, keepdims=True)
    q_ref[...] += jnp.sum(z * z * k1, axis=0, keepdims=True)
    c_ref[...] += jnp.sum(k1, axis=0, keepdims=True)


def _lin2(Ag, h, s1, keep1, W2l, W2r, b2):
    return pl.pallas_call(
        _k_lin2,
        grid=(NI,),
        in_specs=[
            pl.BlockSpec((BI, H1), lambda i: (i, 0)),
            pl.BlockSpec((BI, H1), lambda i: (i, 0)),
            pl.BlockSpec((BI, 1), lambda i: (i, 0)),
            pl.BlockSpec((BI, 1), lambda i: (i, 0)),
            pl.BlockSpec((H1, H2), lambda i: (0, 0)),
            pl.BlockSpec((H1, H2), lambda i: (0, 0)),
            pl.BlockSpec((1, H2), lambda i: (0, 0)),
        ],
        out_specs=[
            pl.BlockSpec((BI, H2), lambda i: (i, 0)),
            pl.BlockSpec((1, H2), lambda i: (0, 0)),
            pl.BlockSpec((1, H2), lambda i: (0, 0)),
            pl.BlockSpec((1, 1), lambda i: (0, 0)),
        ],
        out_shape=[
            jax.ShapeDtypeStruct((NP, H2), F32),
            jax.ShapeDtypeStruct((1, H2), F32),
            jax.ShapeDtypeStruct((1, H2), F32),
            jax.ShapeDtypeStruct((1, 1), F32),
        ],
    )(Ag, h, s1, keep1, W2l, W2r, b2)


def _k_bn2(z_ref, s_ref, q_ref, c_ref, g_ref, be_ref, wl_ref, wr_ref, k1_ref,
           h_ref, v_ref, u_ref):
    cnt = c_ref[...]
    mu = s_ref[...] / cnt
    var = q_ref[...] / cnt - mu * mu
    rstd = lax.rsqrt(var + EPS)
    h = jnp.maximum((z_ref[...] - mu) * rstd * g_ref[...] + be_ref[...], 0.0)
    h_ref[...] = h
    v_ref[...] = jnp.dot(h * k1_ref[...], wl_ref[...], preferred_element_type=F32)
    u_ref[...] = jnp.dot(h, wr_ref[...], preferred_element_type=F32)


def _bn2(Z2, S2, Q2, C2, g2, be2, p2_wl, p2_wr, keep1):
    return pl.pallas_call(
        _k_bn2,
        grid=(NI,),
        in_specs=[
            pl.BlockSpec((BI, H2), lambda i: (i, 0)),
            pl.BlockSpec((1, H2), lambda i: (0, 0)),
            pl.BlockSpec((1, H2), lambda i: (0, 0)),
            pl.BlockSpec((1, 1), lambda i: (0, 0)),
            pl.BlockSpec((1, H2), lambda i: (0, 0)),
            pl.BlockSpec((1, H2), lambda i: (0, 0)),
            pl.BlockSpec((H2, 1), lambda i: (0, 0)),
            pl.BlockSpec((H2, 1), lambda i: (0, 0)),
            pl.BlockSpec((BI, 1), lambda i: (i, 0)),
        ],
        out_specs=[
            pl.BlockSpec((BI, H2), lambda i: (i, 0)),
            pl.BlockSpec((BI, 1), lambda i: (i, 0)),
            pl.BlockSpec((BI, 1), lambda i: (i, 0)),
        ],
        out_shape=[
            jax.ShapeDtypeStruct((NP, H2), F32),
            jax.ShapeDtypeStruct((NP, 1), F32),
            jax.ShapeDtypeStruct((NP, 1), F32),
        ],
    )(Z2, S2, Q2, C2, g2, be2, p2_wl, p2_wr, keep1)


def _k_final(h_ref, s2_ref, k2_ref, b_ref, w_ref, bl_ref,
             p_ref, c_ref, o_ref, *, ni):
    i = pl.program_id(0)

    @pl.when(i == 0)
    def _():
        p_ref[...] = jnp.zeros_like(p_ref)
        c_ref[...] = jnp.zeros_like(c_ref)

    w = k2_ref[...] * jnp.tanh(s2_ref[...])
    h3 = h_ref[...] * w
    grow = lax.broadcasted_iota(jnp.int32, (1, G), 1)
    ind = jnp.where(b_ref[...] == grow, 1.0, 0.0)
    dn = (((0,), (0,)), ((), ()))
    p_ref[...] += lax.dot_general(ind, h3, dn, preferred_element_type=F32)
    c_ref[...] += lax.dot_general(ind, k2_ref[...], dn, preferred_element_type=F32)

    @pl.when(i == ni - 1)
    def _():
        pooled = p_ref[...] / jnp.maximum(c_ref[...], 1.0)
        a = jnp.maximum(pooled, 0.0)
        out = jnp.dot(a, w_ref[...], preferred_element_type=F32) + bl_ref[...]
        lane = lax.broadcasted_iota(jnp.int32, (1, 128), 1)
        valid = lane < 2
        mx = jnp.max(jnp.where(valid, out, -jnp.inf), axis=1, keepdims=True)
        e = jnp.where(valid, jnp.exp(out - mx), 0.0)
        o_ref[...] = e / jnp.sum(e, axis=1, keepdims=True)


def _final(h2, s2, keep2, batch_c, Wlin_p, blin_p):
    outs = pl.pallas_call(
        functools.partial(_k_final, ni=NI),
        grid=(NI,),
        in_specs=[
            pl.BlockSpec((BI, H2), lambda i: (i, 0)),
            pl.BlockSpec((BI, 1), lambda i: (i, 0)),
            pl.BlockSpec((BI, 1), lambda i: (i, 0)),
            pl.BlockSpec((BI, 1), lambda i: (i, 0)),
            pl.BlockSpec((H2, 128), lambda i: (0, 0)),
            pl.BlockSpec((1, 128), lambda i: (0, 0)),
        ],
        out_specs=[
            pl.BlockSpec((G, H2), lambda i: (0, 0)),
            pl.BlockSpec((G, 1), lambda i: (0, 0)),
            pl.BlockSpec((G, 128), lambda i: (0, 0)),
        ],
        out_shape=[
            jax.ShapeDtypeStruct((G, H2), F32),
            jax.ShapeDtypeStruct((G, 1), F32),
            jax.ShapeDtypeStruct((G, 128), F32),
        ],
    )(h2, s2, keep2, batch_c, Wlin_p, blin_p)
    return outs[2]


def kernel(x, edge_index, batch, W1l, W1r, b1, g1, be1, p1_wl, p1_wr, p1_b,
           W2l, W2r, b2, g2, be2, p2_wl, p2_wr, p2_b, Wlin, blin):
    src, dst = edge_index[0], edge_index[1]
    A = jnp.zeros((NP, NP), F32).at[dst, src].add(1.0)

    xp = jnp.zeros((NP, F0), F32).at[:N].set(x)
    xa = jnp.zeros((NP, H1), F32).at[:N, :F0].set(x).at[:, F0].set(1.0)
    batch_c = jnp.concatenate(
        [batch, jnp.full((NP - N,), G, jnp.int32)]).reshape(NP, 1)
    ones_c = jnp.ones((NP, 1), F32)
    valid_c = jnp.concatenate(
        [jnp.ones((N, 1), F32), jnp.zeros((NP - N, 1), F32)])

    C1 = _amm(A, xa, H1)
    Z1, S1, Q1 = _lin1(C1, xp, W1l, W1r, b1.reshape(1, H1))
    h, v1, u1 = _bn1(Z1, S1, Q1, g1.reshape(1, H1), be1.reshape(1, H1),
                     p1_wl, p1_wr)

    s1 = _score(A, v1, u1, ones_c, p1_b.reshape(1, 1))
    keep1 = _topk(s1, jnp.zeros((NP, 1), F32), batch_c, valid_c)

    Ag, _ = _aggr2(A, h, s1, keep1)
    Z2, S2, Q2, C2 = _lin2(Ag, h, s1, keep1, W2l, W2r, b2.reshape(1, H2))
    h2, v2, u2 = _bn2(Z2, S2, Q2, C2, g2.reshape(1, H2), be2.reshape(1, H2),
                      p2_wl, p2_wr, keep1)

    s2 = _score(A, v2, u2, keep1, p2_b.reshape(1, 1))
    keep2 = _topk(s2, s1, batch_c, keep1)

    Wlin_p = jnp.zeros((H2, 128), F32).at[:, :2].set(Wlin)
    blin_p = jnp.zeros((1, 128), F32).at[0, :2].set(blin)
    out = _final(h2, s2, keep2, batch_c, Wlin_p, blin_p)
    return out[:, :2]


# speedup vs baseline: 4.8224x; 1.1377x over previous
import functools

import jax
import jax.numpy as jnp
from jax import lax
from jax.experimental import pallas as pl

N = 10000
NP = 10240
E = 320000
G = 64
F0 = 128
H1 = 256
H2 = 512
BI = 256
BK = 2048
NI = NP // BI
NJ = NP // BK
RATIO = 0.5
EPS = 1e-5
F32 = jnp.float32


def _k_amm(a_ref, x_ref, o_ref, *, nj):
    j = pl.program_id(1)
    part = jnp.dot(a_ref[...], x_ref[...], preferred_element_type=F32)

    @pl.when(j == 0)
    def _():
        o_ref[...] = part

    @pl.when(j > 0)
    def _():
        o_ref[...] += part


def _amm(A, Xa, width):
    return pl.pallas_call(
        functools.partial(_k_amm, nj=NJ),
        grid=(NI, NJ),
        in_specs=[
            pl.BlockSpec((BI, BK), lambda i, j: (i, j)),
            pl.BlockSpec((BK, width), lambda i, j: (j, 0)),
        ],
        out_specs=pl.BlockSpec((BI, width), lambda i, j: (i, 0)),
        out_shape=jax.ShapeDtypeStruct((NP, width), F32),
    )(A, Xa)


def _k_lin1(c_ref, x_ref, wl_ref, wr_ref, b_ref, z_ref, s_ref, q_ref, *, ni):
    i = pl.program_id(0)
    c = c_ref[...]
    aggr = c[:, :F0]
    deg = c[:, F0:F0 + 1]
    aggrn = aggr / jnp.maximum(deg, 1.0)
    z = (jnp.dot(aggrn, wl_ref[...], preferred_element_type=F32)
         + jnp.dot(x_ref[...], wr_ref[...], preferred_element_type=F32)
         + b_ref[...])
    row = i * BI + lax.broadcasted_iota(jnp.int32, (BI, 1), 0)
    z = jnp.where(row < N, z, 0.0)
    z_ref[...] = z

    @pl.when(i == 0)
    def _():
        s_ref[...] = jnp.zeros_like(s_ref)
        q_ref[...] = jnp.zeros_like(q_ref)

    s_ref[...] += jnp.sum(z, axis=0, keepdims=True)
    q_ref[...] += jnp.sum(z * z, axis=0, keepdims=True)


def _lin1(C1, xp, W1l, W1r, b1):
    return pl.pallas_call(
        functools.partial(_k_lin1, ni=NI),
        grid=(NI,),
        in_specs=[
            pl.BlockSpec((BI, H1), lambda i: (i, 0)),
            pl.BlockSpec((BI, F0), lambda i: (i, 0)),
            pl.BlockSpec((F0, H1), lambda i: (0, 0)),
            pl.BlockSpec((F0, H1), lambda i: (0, 0)),
            pl.BlockSpec((1, H1), lambda i: (0, 0)),
        ],
        out_specs=[
            pl.BlockSpec((BI, H1), lambda i: (i, 0)),
            pl.BlockSpec((1, H1), lambda i: (0, 0)),
            pl.BlockSpec((1, H1), lambda i: (0, 0)),
        ],
        out_shape=[
            jax.ShapeDtypeStruct((NP, H1), F32),
            jax.ShapeDtypeStruct((1, H1), F32),
            jax.ShapeDtypeStruct((1, H1), F32),
        ],
    )(C1, xp, W1l, W1r, b1)


def _k_bn1(z_ref, s_ref, q_ref, g_ref, be_ref, wl_ref, wr_ref,
           h_ref, v_ref, u_ref):
    mu = s_ref[...] / N
    var = q_ref[...] / N - mu * mu
    rstd = lax.rsqrt(var + EPS)
    h = jnp.maximum((z_ref[...] - mu) * rstd * g_ref[...] + be_ref[...], 0.0)
    h_ref[...] = h
    v_ref[...] = jnp.dot(h, wl_ref[...], preferred_element_type=F32)
    u_ref[...] = jnp.dot(h, wr_ref[...], preferred_element_type=F32)


def _bn1(Z1, S1, Q1, g1, be1, p1_wl, p1_wr):
    return pl.pallas_call(
        _k_bn1,
        grid=(NI,),
        in_specs=[
            pl.BlockSpec((BI, H1), lambda i: (i, 0)),
            pl.BlockSpec((1, H1), lambda i: (0, 0)),
            pl.BlockSpec((1, H1), lambda i: (0, 0)),
            pl.BlockSpec((1, H1), lambda i: (0, 0)),
            pl.BlockSpec((1, H1), lambda i: (0, 0)),
            pl.BlockSpec((H1, 1), lambda i: (0, 0)),
            pl.BlockSpec((H1, 1), lambda i: (0, 0)),
        ],
        out_specs=[
            pl.BlockSpec((BI, H1), lambda i: (i, 0)),
            pl.BlockSpec((BI, 1), lambda i: (i, 0)),
            pl.BlockSpec((BI, 1), lambda i: (i, 0)),
        ],
        out_shape=[
            jax.ShapeDtypeStruct((NP, H1), F32),
            jax.ShapeDtypeStruct((NP, 1), F32),
            jax.ShapeDtypeStruct((NP, 1), F32),
        ],
    )(Z1, S1, Q1, g1, be1, p1_wl, p1_wr)


def _k_matvec(a_ref, v_ref, u_ref, m_ref, b_ref, o_ref, *, nj):
    j = pl.program_id(1)
    part = jnp.dot(a_ref[...], v_ref[...], preferred_element_type=F32)

    @pl.when(j == 0)
    def _():
        o_ref[...] = part

    @pl.when(j > 0)
    def _():
        o_ref[...] += part

    @pl.when(j == nj - 1)
    def _():
        o_ref[...] = m_ref[...] * o_ref[...] + u_ref[...] + b_ref[...]


def _score(A, v, u, m, b):
    return pl.pallas_call(
        functools.partial(_k_matvec, nj=NJ),
        grid=(NI, NJ),
        in_specs=[
            pl.BlockSpec((BI, BK), lambda i, j: (i, j)),
            pl.BlockSpec((BK, 1), lambda i, j: (j, 0)),
            pl.BlockSpec((BI, 1), lambda i, j: (i, 0)),
            pl.BlockSpec((BI, 1), lambda i, j: (i, 0)),
            pl.BlockSpec((1, 1), lambda i, j: (0, 0)),
        ],
        out_specs=pl.BlockSpec((BI, 1), lambda i, j: (i, 0)),
        out_shape=jax.ShapeDtypeStruct((NP, 1), F32),
    )(A, v, u, m, b)


def _k_topk(sc_ref, tc_ref, bc_ref, mc_ref, sr_ref, tr_ref, br_ref, mr_ref,
            rank_ref, cnt_ref, keep_ref, *, nj):
    i = pl.program_id(0)
    j = pl.program_id(1)

    @pl.when(j == 0)
    def _():
        rank_ref[...] = jnp.zeros_like(rank_ref)
        cnt_ref[...] = jnp.zeros_like(cnt_ref)

    bi = bc_ref[...]
    bj = br_ref[0]
    overlap = (jnp.min(bj) <= jnp.max(bi)) & (jnp.max(bj) >= jnp.min(bi))

    @pl.when(overlap)
    def _():
        si = sc_ref[...]
        sj = sr_ref[0]
        ti = tc_ref[...]
        tj = tr_ref[0]
        mj = mr_ref[0] > 0.0
        ii = i * BI + lax.broadcasted_iota(jnp.int32, (BI, 1), 0)
        jj = j * BK + lax.broadcasted_iota(jnp.int32, (1, BK), 1)
        sg = (bi == bj) & mj
        gt = sj > si
        eq = sj == si
        tgt = tj > ti
        teq = tj == ti
        prec = gt | (eq & (tgt | (teq & (jj < ii))))
        rank_ref[...] += jnp.sum(jnp.where(sg & prec, 1.0, 0.0), axis=1, keepdims=True)
        cnt_ref[...] += jnp.sum(jnp.where(sg, 1.0, 0.0), axis=1, keepdims=True)

    @pl.when(j == nj - 1)
    def _():
        k = jnp.ceil(RATIO * cnt_ref[...])
        keep_ref[...] = jnp.where((mc_ref[...] > 0.0) & (rank_ref[...] < k),
                                  1.0, 0.0)


def _topk(s, t, batch_c, m):
    s_r = s.reshape(NJ, 1, BK)
    t_r = t.reshape(NJ, 1, BK)
    b_r = batch_c.reshape(NJ, 1, BK)
    m_r = m.reshape(NJ, 1, BK)
    col = lambda i, j: (i, 0)
    row = lambda i, j: (j, 0, 0)
    outs = pl.pallas_call(
        functools.partial(_k_topk, nj=NJ),
        grid=(NI, NJ),
        in_specs=[
            pl.BlockSpec((BI, 1), col),
            pl.BlockSpec((BI, 1), col),
            pl.BlockSpec((BI, 1), col),
            pl.BlockSpec((BI, 1), col),
            pl.BlockSpec((1, 1, BK), row),
            pl.BlockSpec((1, 1, BK), row),
            pl.BlockSpec((1, 1, BK), row),
            pl.BlockSpec((1, 1, BK), row),
        ],
        out_specs=[
            pl.BlockSpec((BI, 1), col),
            pl.BlockSpec((BI, 1), col),
            pl.BlockSpec((BI, 1), col),
        ],
        out_shape=[
            jax.ShapeDtypeStruct((NP, 1), F32),
            jax.ShapeDtypeStruct((NP, 1), F32),
            jax.ShapeDtypeStruct((NP, 1), F32),
        ],
    )(s, t, batch_c, m, s_r, t_r, b_r, m_r)
    return outs[2]


def _k_aggr2(a_ref, h_ref, s1_ref, k1j_ref, k1r_ref, k1i_ref, ag_ref, dm_ref, *, nj):
    j = pl.program_id(1)
    a = a_ref[...]
    w = k1j_ref[...] * jnp.tanh(s1_ref[...])
    hm = h_ref[...] * w
    part = jnp.dot(a, hm, preferred_element_type=F32)
    dpart = jnp.sum(a * k1r_ref[0], axis=1, keepdims=True)

    @pl.when(j == 0)
    def _():
        ag_ref[...] = part
        dm_ref[...] = dpart

    @pl.when(j > 0)
    def _():
        ag_ref[...] += part
        dm_ref[...] += dpart

    @pl.when(j == nj - 1)
    def _():
        mi = k1i_ref[...]
        degm = mi * dm_ref[...]
        ag_ref[...] = mi * ag_ref[...] / jnp.maximum(degm, 1.0)


def _aggr2(A, h, s1, keep1):
    k1_r = keep1.reshape(NJ, 1, BK)
    return pl.pallas_call(
        functools.partial(_k_aggr2, nj=NJ),
        grid=(NI, NJ),
        in_specs=[
            pl.BlockSpec((BI, BK), lambda i, j: (i, j)),
            pl.BlockSpec((BK, H1), lambda i, j: (j, 0)),
            pl.BlockSpec((BK, 1), lambda i, j: (j, 0)),
            pl.BlockSpec((BK, 1), lambda i, j: (j, 0)),
            pl.BlockSpec((1, 1, BK), lambda i, j: (j, 0, 0)),
            pl.BlockSpec((BI, 1), lambda i, j: (i, 0)),
        ],
        out_specs=[
            pl.BlockSpec((BI, H1), lambda i, j: (i, 0)),
            pl.BlockSpec((BI, 1), lambda i, j: (i, 0)),
        ],
        out_shape=[
            jax.ShapeDtypeStruct((NP, H1), F32),
            jax.ShapeDtypeStruct((NP, 1), F32),
        ],
    )(A, h, s1, keep1, k1_r, keep1)


def _k_lin2(ag_ref, h_ref, s1_ref, k1_ref, wl_ref, wr_ref, b_ref,
            z_ref, s_ref, q_ref, c_ref):
    i = pl.program_id(0)
    k1 = k1_ref[...]
    hm = h_ref[...] * (k1 * jnp.tanh(s1_ref[...]))
    z = (jnp.dot(ag_ref[...], wl_ref[...], preferred_element_type=F32)
         + jnp.dot(hm, wr_ref[...], preferred_element_type=F32)
         + b_ref[...])
    z_ref[...] = z

    @pl.when(i == 0)
    def _():
        s_ref[...] = jnp.zeros_like(s_ref)
        q_ref[...] = jnp.zeros_like(q_ref)
        c_ref[...] = jnp.zeros_like(c_ref)

    s_ref[...] += jnp.sum(z * k1, axis=0, keepdims=True)
    q_ref[...] += jnp.sum(z * z * k1, axis=0, keepdims=True)
    c_ref[...] += jnp.sum(k1, axis=0, keepdims=True)


def _lin2(Ag, h, s1, keep1, W2l, W2r, b2):
    return pl.pallas_call(
        _k_lin2,
        grid=(NI,),
        in_specs=[
            pl.BlockSpec((BI, H1), lambda i: (i, 0)),
            pl.BlockSpec((BI, H1), lambda i: (i, 0)),
            pl.BlockSpec((BI, 1), lambda i: (i, 0)),
            pl.BlockSpec((BI, 1), lambda i: (i, 0)),
            pl.BlockSpec((H1, H2), lambda i: (0, 0)),
            pl.BlockSpec((H1, H2), lambda i: (0, 0)),
            pl.BlockSpec((1, H2), lambda i: (0, 0)),
        ],
        out_specs=[
            pl.BlockSpec((BI, H2), lambda i: (i, 0)),
            pl.BlockSpec((1, H2), lambda i: (0, 0)),
            pl.BlockSpec((1, H2), lambda i: (0, 0)),
            pl.BlockSpec((1, 1), lambda i: (0, 0)),
        ],
        out_shape=[
            jax.ShapeDtypeStruct((NP, H2), F32),
            jax.ShapeDtypeStruct((1, H2), F32),
            jax.ShapeDtypeStruct((1, H2), F32),
            jax.ShapeDtypeStruct((1, 1), F32),
        ],
    )(Ag, h, s1, keep1, W2l, W2r, b2)


def _k_bn2(z_ref, s_ref, q_ref, c_ref, g_ref, be_ref, wl_ref, wr_ref, k1_ref,
           h_ref, v_ref, u_ref):
    cnt = c_ref[...]
    mu = s_ref[...] / cnt
    var = q_ref[...] / cnt - mu * mu
    rstd = lax.rsqrt(var + EPS)
    h = jnp.maximum((z_ref[...] - mu) * rstd * g_ref[...] + be_ref[...], 0.0)
    h_ref[...] = h
    v_ref[...] = jnp.dot(h * k1_ref[...], wl_ref[...], preferred_element_type=F32)
    u_ref[...] = jnp.dot(h, wr_ref[...], preferred_element_type=F32)


def _bn2(Z2, S2, Q2, C2, g2, be2, p2_wl, p2_wr, keep1):
    return pl.pallas_call(
        _k_bn2,
        grid=(NI,),
        in_specs=[
            pl.BlockSpec((BI, H2), lambda i: (i, 0)),
            pl.BlockSpec((1, H2), lambda i: (0, 0)),
            pl.BlockSpec((1, H2), lambda i: (0, 0)),
            pl.BlockSpec((1, 1), lambda i: (0, 0)),
            pl.BlockSpec((1, H2), lambda i: (0, 0)),
            pl.BlockSpec((1, H2), lambda i: (0, 0)),
            pl.BlockSpec((H2, 1), lambda i: (0, 0)),
            pl.BlockSpec((H2, 1), lambda i: (0, 0)),
            pl.BlockSpec((BI, 1), lambda i: (i, 0)),
        ],
        out_specs=[
            pl.BlockSpec((BI, H2), lambda i: (i, 0)),
            pl.BlockSpec((BI, 1), lambda i: (i, 0)),
            pl.BlockSpec((BI, 1), lambda i: (i, 0)),
        ],
        out_shape=[
            jax.ShapeDtypeStruct((NP, H2), F32),
            jax.ShapeDtypeStruct((NP, 1), F32),
            jax.ShapeDtypeStruct((NP, 1), F32),
        ],
    )(Z2, S2, Q2, C2, g2, be2, p2_wl, p2_wr, keep1)


def _k_final(h_ref, s2_ref, k2_ref, b_ref, w_ref, bl_ref,
             p_ref, c_ref, o_ref, *, ni):
    i = pl.program_id(0)

    @pl.when(i == 0)
    def _():
        p_ref[...] = jnp.zeros_like(p_ref)
        c_ref[...] = jnp.zeros_like(c_ref)

    w = k2_ref[...] * jnp.tanh(s2_ref[...])
    h3 = h_ref[...] * w
    grow = lax.broadcasted_iota(jnp.int32, (1, G), 1)
    ind = jnp.where(b_ref[...] == grow, 1.0, 0.0)
    dn = (((0,), (0,)), ((), ()))
    p_ref[...] += lax.dot_general(ind, h3, dn, preferred_element_type=F32)
    c_ref[...] += lax.dot_general(ind, k2_ref[...], dn, preferred_element_type=F32)

    @pl.when(i == ni - 1)
    def _():
        pooled = p_ref[...] / jnp.maximum(c_ref[...], 1.0)
        a = jnp.maximum(pooled, 0.0)
        out = jnp.dot(a, w_ref[...], preferred_element_type=F32) + bl_ref[...]
        lane = lax.broadcasted_iota(jnp.int32, (1, 128), 1)
        valid = lane < 2
        mx = jnp.max(jnp.where(valid, out, -jnp.inf), axis=1, keepdims=True)
        e = jnp.where(valid, jnp.exp(out - mx), 0.0)
        o_ref[...] = e / jnp.sum(e, axis=1, keepdims=True)


def _final(h2, s2, keep2, batch_c, Wlin_p, blin_p):
    outs = pl.pallas_call(
        functools.partial(_k_final, ni=NI),
        grid=(NI,),
        in_specs=[
            pl.BlockSpec((BI, H2), lambda i: (i, 0)),
            pl.BlockSpec((BI, 1), lambda i: (i, 0)),
            pl.BlockSpec((BI, 1), lambda i: (i, 0)),
            pl.BlockSpec((BI, 1), lambda i: (i, 0)),
            pl.BlockSpec((H2, 128), lambda i: (0, 0)),
            pl.BlockSpec((1, 128), lambda i: (0, 0)),
        ],
        out_specs=[
            pl.BlockSpec((G, H2), lambda i: (0, 0)),
            pl.BlockSpec((G, 1), lambda i: (0, 0)),
            pl.BlockSpec((G, 128), lambda i: (0, 0)),
        ],
        out_shape=[
            jax.ShapeDtypeStruct((G, H2), F32),
            jax.ShapeDtypeStruct((G, 1), F32),
            jax.ShapeDtypeStruct((G, 128), F32),
        ],
    )(h2, s2, keep2, batch_c, Wlin_p, blin_p)
    return outs[2]


def kernel(x, edge_index, batch, W1l, W1r, b1, g1, be1, p1_wl, p1_wr, p1_b,
           W2l, W2r, b2, g2, be2, p2_wl, p2_wr, p2_b, Wlin, blin):
    src, dst = edge_index[0], edge_index[1]
    A = jnp.zeros((NP, NP), F32).at[dst, src].add(1.0)

    xp = jnp.zeros((NP, F0), F32).at[:N].set(x)
    xa = jnp.zeros((NP, H1), F32).at[:N, :F0].set(x).at[:, F0].set(1.0)
    batch_c = jnp.concatenate(
        [batch, jnp.full((NP - N,), G, jnp.int32)]).reshape(NP, 1)
    ones_c = jnp.ones((NP, 1), F32)
    valid_c = jnp.concatenate(
        [jnp.ones((N, 1), F32), jnp.zeros((NP - N, 1), F32)])

    C1 = _amm(A, xa, H1)
    Z1, S1, Q1 = _lin1(C1, xp, W1l, W1r, b1.reshape(1, H1))
    h, v1, u1 = _bn1(Z1, S1, Q1, g1.reshape(1, H1), be1.reshape(1, H1),
                     p1_wl, p1_wr)

    s1 = _score(A, v1, u1, ones_c, p1_b.reshape(1, 1))
    keep1 = _topk(s1, jnp.zeros((NP, 1), F32), batch_c, valid_c)

    Ag, _ = _aggr2(A, h, s1, keep1)
    Z2, S2, Q2, C2 = _lin2(Ag, h, s1, keep1, W2l, W2r, b2.reshape(1, H2))
    h2, v2, u2 = _bn2(Z2, S2, Q2, C2, g2.reshape(1, H2), be2.reshape(1, H2),
                      p2_wl, p2_wr, keep1)

    s2 = _score(A, v2, u2, keep1, p2_b.reshape(1, 1))
    keep2 = _topk(s2, s1, batch_c, keep1)

    Wlin_p = jnp.zeros((H2, 128), F32).at[:, :2].set(Wlin)
    blin_p = jnp.zeros((1, 128), F32).at[0, :2].set(blin)
    out = _final(h2, s2, keep2, batch_c, Wlin_p, blin_p)
    return out[:, :2]
